# baseline (speedup 1.0000x reference)
import functools

import jax
import jax.numpy as jnp
from jax import lax
from jax.experimental import pallas as pl
from jax.experimental.pallas import tpu as pltpu
from jax.experimental.pallas import tpu_sc as plsc

_D_MODEL = 1024
_D_EMBED = 128
_N_CHR = 32
_ECC_BITS = 32
_K = 32
_SEQ = 2048
_R = 256
_NB = _SEQ // _R

_NC = 2
_NS = 16
_NW = _NC * _NS
_NROW = 4 * _SEQ
_RPW = _NROW // _NW
_CH = 8
_NCHUNK = _RPW // _CH
_NVEC = _SEQ // 16
_CAP = 1056
_NEG = jnp.float32(-3e9)


def _tc_kernel(x_ref, we_ref, be_ref, wd_ref, wc_ref, bc_ref,
               wp_ref, bp_ref, w1_ref, b1_ref, w2_ref, b2_ref,
               emb_ref, met_ref, chr_ref, ecc_ref, sim_ref,
               embn_ref):
    p = pl.program_id(0)
    b = pl.program_id(1)
    s = pl.program_id(2)
    row0 = (b * _NB + s) * _R

    @pl.when(p == 0)
    def _phase0():
        x = x_ref[0]
        emb = jnp.dot(x, we_ref[...], preferred_element_type=jnp.float32) + be_ref[...]
        nrm = jnp.sqrt(jnp.sum(emb * emb, axis=1, keepdims=True)) + 1e-8
        embn_ref[pl.ds(row0, _R), :] = emb / nrm

    @pl.when(p == 1)
    def _phase1():
        x = x_ref[0]
        emb = jnp.dot(x, we_ref[...], preferred_element_type=jnp.float32) + be_ref[...]
        emb_ref[0] = emb
        met_ref[0] = jnp.dot(x, wd_ref[...], preferred_element_type=jnp.float32)
        chr_ref[0] = jnp.dot(x, wc_ref[...], preferred_element_type=jnp.float32) + bc_ref[...]
        pr = jnp.dot(x, wp_ref[...], preferred_element_type=jnp.float32) + bp_ref[...]
        h = jnp.tanh(jnp.dot(pr, w1_ref[...], preferred_element_type=jnp.float32) + b1_ref[...])
        ecc_ref[0] = jax.nn.sigmoid(
            jnp.dot(h, w2_ref[...], preferred_element_type=jnp.float32) + b2_ref[...])

        q = embn_ref[pl.ds(row0, _R), :]
        km = embn_ref[pl.ds(b * _SEQ, _SEQ), :]
        sim = jax.lax.dot_general(q, km, (((1,), (1,)), ((), ())),
                                  preferred_element_type=jnp.float32)
        rows = jax.lax.broadcasted_iota(jnp.int32, (_R, _SEQ), 0) + s * _R
        cols = jax.lax.broadcasted_iota(jnp.int32, (_R, _SEQ), 1)
        sim_ref[0] = jnp.where(rows == cols, jnp.float32(-1e9), sim)


def _rotate16(v, k):
    idx = lax.rem(lax.iota(jnp.int32, 16) + k, jnp.int32(16))
    dn = lax.GatherDimensionNumbers(offset_dims=(), collapsed_slice_dims=(0,),
                                    start_index_map=(0,))
    return lax.gather(v, idx.reshape(16, 1), dn, (1,),
                      mode=lax.GatherScatterMode.PROMISE_IN_BOUNDS)


def _splat_min(v):
    for k in (8, 4, 2, 1):
        v = jnp.minimum(v, _rotate16(v, k))
    return v


def _splat_max(v):
    for k in (8, 4, 2, 1):
        v = jnp.maximum(v, _rotate16(v, k))
    return v


def _sc_topk(sim_ref, outs_ref, outi_ref, chunk_ref, mbuf_ref,
             osc_ref, oix_ref):
    cid = lax.axis_index("c")
    sid = lax.axis_index("s")
    wid = sid * _NC + cid
    base = wid * _RPW

    iota = lax.iota(jnp.int32, 16)
    iotaf = iota.astype(jnp.float32)
    big_f = jnp.full((16,), jnp.float32(1e9))
    negv = jnp.full((16,), _NEG)

    def chunk_body(c, _):
        pltpu.sync_copy(sim_ref.at[pl.ds(base + c * _CH, _CH)], chunk_ref)

        def row_body(rr, _):
            def p1(i, _):
                v = chunk_ref[rr, pl.ds(i * 16, 16)]
                mbuf_ref[pl.ds(i, 16)] = _splat_max(v)
                return 0
            lax.fori_loop(0, _NVEC, p1, 0)

            def extract(k, _):
                def fmax(j, acc):
                    return jnp.maximum(acc, mbuf_ref[pl.ds(j * 16, 16)])
                macc = lax.fori_loop(0, _NVEC // 16, fmax, negv)
                mkv = _splat_max(macc)

                def floc(j, vacc):
                    mv = mbuf_ref[pl.ds(j * 16, 16)]
                    loc = mv == mkv
                    cand = jnp.where(loc, iotaf + (j * 16).astype(jnp.float32),
                                     big_f)
                    return jnp.minimum(vacc, cand)
                vacc = lax.fori_loop(0, _NVEC // 16, floc, big_f)
                vecid = _splat_min(vacc)[0].astype(jnp.int32)

                w = chunk_ref[rr, pl.ds(vecid * 16, 16)]
                loc = w == mkv
                aminv = _splat_min(jnp.where(loc, iotaf, big_f)) \
                    + jnp.float32(16) * vecid.astype(jnp.float32)
                w2 = jnp.where(loc, negv, w)
                chunk_ref[rr, pl.ds(vecid * 16, 16)] = w2

                mwin = mbuf_ref[pl.ds(vecid, 16)]
                mbuf_ref[pl.ds(vecid, 16)] = jnp.where(
                    iota == 0, _splat_max(w2), mwin)

                opos = rr * _K + k
                osc_ref[pl.ds(opos, 16)] = mkv
                oix_ref[pl.ds(opos, 16)] = aminv.astype(jnp.int32)
                return 0
            lax.fori_loop(0, _K, extract, 0)
            return 0
        lax.fori_loop(0, _CH, row_body, 0)

        o0 = (base + c * _CH) * _K
        pltpu.sync_copy(osc_ref.at[pl.ds(0, _CH * _K)],
                        outs_ref.at[pl.ds(o0, _CH * _K)])
        pltpu.sync_copy(oix_ref.at[pl.ds(0, _CH * _K)],
                        outi_ref.at[pl.ds(o0, _CH * _K)])
        return 0
    lax.fori_loop(0, _NCHUNK, chunk_body, 0)


def kernel(x, W_embed, b_embed, W_diag, W_chr, b_chr,
           W_ecc_proj, b_ecc_proj, W_e1, b_e1, W_e2, b_e2):
    B, S, D = x.shape
    f32 = jnp.float32

    be = b_embed.reshape(1, -1)
    bc = b_chr.reshape(1, -1)
    bp = b_ecc_proj.reshape(1, -1)
    b1 = b_e1.reshape(1, -1)
    b2 = b_e2.reshape(1, -1)

    full = lambda shp: pl.BlockSpec(shp, lambda p, b, s: (0,) * len(shp))
    blk = lambda w: pl.BlockSpec((1, _R, w), lambda p, b, s: (b, s, 0))
    emb, met, chrs, ecc, sim = pl.pallas_call(
        _tc_kernel,
        grid=(2, B, _NB),
        in_specs=[
            pl.BlockSpec((1, _R, D), lambda p, b, s: (b, s, 0)),
            full((D, _D_EMBED)), full((1, _D_EMBED)),
            full((D, D)),
            full((D, _N_CHR)), full((1, _N_CHR)),
            full((D, _ECC_BITS)), full((1, _ECC_BITS)),
            full((_ECC_BITS, 2 * _ECC_BITS)), full((1, 2 * _ECC_BITS)),
            full((2 * _ECC_BITS, _ECC_BITS)), full((1, _ECC_BITS)),
        ],
        out_specs=[
            blk(_D_EMBED), blk(D), blk(_N_CHR), blk(_ECC_BITS), blk(_SEQ),
        ],
        out_shape=[
            jax.ShapeDtypeStruct((B, S, _D_EMBED), f32),
            jax.ShapeDtypeStruct((B, S, D), f32),
            jax.ShapeDtypeStruct((B, S, _N_CHR), f32),
            jax.ShapeDtypeStruct((B, S, _ECC_BITS), f32),
            jax.ShapeDtypeStruct((B, S, _SEQ), f32),
        ],
        scratch_shapes=[pltpu.VMEM((B * S, _D_EMBED), f32)],
        compiler_params=pltpu.CompilerParams(
            dimension_semantics=("arbitrary", "arbitrary", "arbitrary")),
    )(x, W_embed, be, W_diag, W_chr, bc, W_ecc_proj, bp, W_e1, b1, W_e2, b2)

    sim2 = sim.reshape(B * S, S)
    mesh = plsc.VectorSubcoreMesh(core_axis_name="c", subcore_axis_name="s")
    scores_flat, idx_flat = pl.kernel(
        _sc_topk,
        out_type=[
            jax.ShapeDtypeStruct((_NROW * _K,), f32),
            jax.ShapeDtypeStruct((_NROW * _K,), jnp.int32),
        ],
        mesh=mesh,
        scratch_types=[
            pltpu.VMEM((_CH, _SEQ), f32),
            pltpu.VMEM((_NVEC + 32,), f32),
            pltpu.VMEM((_CH * _K + 16,), f32),
            pltpu.VMEM((_CH * _K + 16,), jnp.int32),
        ],
    )(sim2)

    scores = scores_flat.reshape(B, S, _K)
    idx = idx_flat.reshape(B, S, _K)
    minh = scores[..., :_K // 2]
    maxh = -scores[..., _K // 2:]
    return (emb, met, chrs, scores, idx, minh, maxh, ecc)


# speedup vs baseline: 2.1535x; 2.1535x over previous
import functools

import jax
import jax.numpy as jnp
from jax import lax
from jax.experimental import pallas as pl
from jax.experimental.pallas import tpu as pltpu
from jax.experimental.pallas import tpu_sc as plsc

_D_MODEL = 1024
_D_EMBED = 128
_N_CHR = 32
_ECC_BITS = 32
_K = 32
_SEQ = 2048
_R = 256
_NB = _SEQ // _R

_NC = 2
_NS = 16
_NW = _NC * _NS
_NROW = 4 * _SEQ
_NSC = 2560
_NTC = _NROW - _NSC
_RPW = _NSC // _NW
_CH = 8
_NCHUNK = _RPW // _CH
_NVEC = _SEQ // 16
_NEG = jnp.float32(-3e9)


def _tc_kernel(x_ref, we_ref, be_ref, wd_ref, wc_ref, bc_ref,
               wp_ref, bp_ref, w1_ref, b1_ref, w2_ref, b2_ref,
               emb_ref, met_ref, chr_ref, ecc_ref, sim_ref,
               embn_ref):
    p = pl.program_id(0)
    b = pl.program_id(1)
    s = pl.program_id(2)
    row0 = (b * _NB + s) * _R

    @pl.when(p == 0)
    def _phase0():
        x = x_ref[0]
        emb = jnp.dot(x, we_ref[...], preferred_element_type=jnp.float32) + be_ref[...]
        nrm = jnp.sqrt(jnp.sum(emb * emb, axis=1, keepdims=True)) + 1e-8
        embn_ref[pl.ds(row0, _R), :] = emb / nrm

    @pl.when(p == 1)
    def _phase1():
        x = x_ref[0]
        emb = jnp.dot(x, we_ref[...], preferred_element_type=jnp.float32) + be_ref[...]
        emb_ref[0] = emb
        met_ref[0] = jnp.dot(x, wd_ref[...], preferred_element_type=jnp.float32)
        chr_ref[0] = jnp.dot(x, wc_ref[...], preferred_element_type=jnp.float32) + bc_ref[...]
        pr = jnp.dot(x, wp_ref[...], preferred_element_type=jnp.float32) + bp_ref[...]
        h = jnp.tanh(jnp.dot(pr, w1_ref[...], preferred_element_type=jnp.float32) + b1_ref[...])
        ecc_ref[0] = jax.nn.sigmoid(
            jnp.dot(h, w2_ref[...], preferred_element_type=jnp.float32) + b2_ref[...])

        q = embn_ref[pl.ds(row0, _R), :]
        km = embn_ref[pl.ds(b * _SEQ, _SEQ), :]
        sim = jax.lax.dot_general(q, km, (((1,), (1,)), ((), ())),
                                  preferred_element_type=jnp.float32)
        rows = jax.lax.broadcasted_iota(jnp.int32, (_R, _SEQ), 0) + s * _R
        cols = jax.lax.broadcasted_iota(jnp.int32, (_R, _SEQ), 1)
        sim_ref[0] = jnp.where(rows == cols, jnp.float32(-1e9), sim)


def _rotate16(v, k):
    idx = lax.rem(lax.iota(jnp.int32, 16) + k, jnp.int32(16))
    dn = lax.GatherDimensionNumbers(offset_dims=(), collapsed_slice_dims=(0,),
                                    start_index_map=(0,))
    return lax.gather(v, idx.reshape(16, 1), dn, (1,),
                      mode=lax.GatherScatterMode.PROMISE_IN_BOUNDS)


def _splat_min(v):
    for k in (8, 4, 2, 1):
        v = jnp.minimum(v, _rotate16(v, k))
    return v


def _splat_max(v):
    for k in (8, 4, 2, 1):
        v = jnp.maximum(v, _rotate16(v, k))
    return v


def _sc_topk(sim_ref, outs_ref, outi_ref, chunk_ref, mbuf_ref,
             osc_ref, oix_ref):
    cid = lax.axis_index("c")
    sid = lax.axis_index("s")
    wid = sid * _NC + cid
    base = _NTC + wid * _RPW

    iota = lax.iota(jnp.int32, 16)
    iotaf = iota.astype(jnp.float32)
    big_f = jnp.full((16,), jnp.float32(1e9))
    negv = jnp.full((16,), _NEG)

    def chunk_body(c, _):
        pltpu.sync_copy(sim_ref.at[pl.ds(base + c * _CH, _CH)], chunk_ref)

        def row_body(rr, _):
            def p1(i, _):
                v = chunk_ref[rr, pl.ds(i * 16, 16)]
                mbuf_ref[pl.ds(i, 16)] = _splat_max(v)
                return 0
            lax.fori_loop(0, _NVEC, p1, 0)

            def extract(k, _):
                def fmax(j, acc):
                    return jnp.maximum(acc, mbuf_ref[pl.ds(j * 16, 16)])
                macc = lax.fori_loop(0, _NVEC // 16, fmax, negv)
                mkv = _splat_max(macc)

                def floc(j, vacc):
                    mv = mbuf_ref[pl.ds(j * 16, 16)]
                    loc = mv == mkv
                    cand = jnp.where(loc, iotaf + (j * 16).astype(jnp.float32),
                                     big_f)
                    return jnp.minimum(vacc, cand)
                vacc = lax.fori_loop(0, _NVEC // 16, floc, big_f)
                vecid = _splat_min(vacc)[0].astype(jnp.int32)

                w = chunk_ref[rr, pl.ds(vecid * 16, 16)]
                loc = w == mkv
                aminv = _splat_min(jnp.where(loc, iotaf, big_f)) \
                    + jnp.float32(16) * vecid.astype(jnp.float32)
                w2 = jnp.where(loc, negv, w)
                chunk_ref[rr, pl.ds(vecid * 16, 16)] = w2

                mwin = mbuf_ref[pl.ds(vecid, 16)]
                mbuf_ref[pl.ds(vecid, 16)] = jnp.where(
                    iota == 0, _splat_max(w2), mwin)

                opos = rr * _K + k
                osc_ref[pl.ds(opos, 16)] = mkv
                oix_ref[pl.ds(opos, 16)] = aminv.astype(jnp.int32)
                return 0
            lax.fori_loop(0, _K, extract, 0)
            return 0
        lax.fori_loop(0, _CH, row_body, 0)

        o0 = (wid * _RPW + c * _CH) * _K
        pltpu.sync_copy(osc_ref.at[pl.ds(0, _CH * _K)],
                        outs_ref.at[pl.ds(o0, _CH * _K)])
        pltpu.sync_copy(oix_ref.at[pl.ds(0, _CH * _K)],
                        outi_ref.at[pl.ds(o0, _CH * _K)])
        return 0
    lax.fori_loop(0, _NCHUNK, chunk_body, 0)


def _tc_topk(sim_ref, scores_ref, idx_ref):
    work = sim_ref[...]
    colsf = jax.lax.broadcasted_iota(jnp.int32, (_R, _SEQ), 1).astype(jnp.float32)
    s_list, i_list = [], []
    for _ in range(_K):
        m = jnp.max(work, axis=1, keepdims=True)
        loc = work == m
        amin = jnp.min(jnp.where(loc, colsf, jnp.float32(_SEQ)),
                       axis=1, keepdims=True)
        s_list.append(m)
        i_list.append(amin)
        work = jnp.where(loc, jnp.float32(-2e9), work)
    scores_ref[...] = jnp.concatenate(s_list, axis=1)
    idx_ref[...] = jnp.concatenate(i_list, axis=1).astype(jnp.int32)


def kernel(x, W_embed, b_embed, W_diag, W_chr, b_chr,
           W_ecc_proj, b_ecc_proj, W_e1, b_e1, W_e2, b_e2):
    B, S, D = x.shape
    f32 = jnp.float32

    be = b_embed.reshape(1, -1)
    bc = b_chr.reshape(1, -1)
    bp = b_ecc_proj.reshape(1, -1)
    b1 = b_e1.reshape(1, -1)
    b2 = b_e2.reshape(1, -1)

    full = lambda shp: pl.BlockSpec(shp, lambda p, b, s: (0,) * len(shp))
    blk = lambda w: pl.BlockSpec((1, _R, w), lambda p, b, s: (b, s, 0))
    emb, met, chrs, ecc, sim = pl.pallas_call(
        _tc_kernel,
        grid=(2, B, _NB),
        in_specs=[
            pl.BlockSpec((1, _R, D), lambda p, b, s: (b, s, 0)),
            full((D, _D_EMBED)), full((1, _D_EMBED)),
            full((D, D)),
            full((D, _N_CHR)), full((1, _N_CHR)),
            full((D, _ECC_BITS)), full((1, _ECC_BITS)),
            full((_ECC_BITS, 2 * _ECC_BITS)), full((1, 2 * _ECC_BITS)),
            full((2 * _ECC_BITS, _ECC_BITS)), full((1, _ECC_BITS)),
        ],
        out_specs=[
            blk(_D_EMBED), blk(D), blk(_N_CHR), blk(_ECC_BITS), blk(_SEQ),
        ],
        out_shape=[
            jax.ShapeDtypeStruct((B, S, _D_EMBED), f32),
            jax.ShapeDtypeStruct((B, S, D), f32),
            jax.ShapeDtypeStruct((B, S, _N_CHR), f32),
            jax.ShapeDtypeStruct((B, S, _ECC_BITS), f32),
            jax.ShapeDtypeStruct((B, S, _SEQ), f32),
        ],
        scratch_shapes=[pltpu.VMEM((B * S, _D_EMBED), f32)],
        compiler_params=pltpu.CompilerParams(
            dimension_semantics=("arbitrary", "arbitrary", "arbitrary")),
    )(x, W_embed, be, W_diag, W_chr, bc, W_ecc_proj, bp, W_e1, b1, W_e2, b2)

    sim2 = sim.reshape(B * S, S)

    tc_scores, tc_idx = pl.pallas_call(
        _tc_topk,
        grid=(_NTC // _R,),
        in_specs=[pl.BlockSpec((_R, _SEQ), lambda i: (i, 0))],
        out_specs=[pl.BlockSpec((_R, _K), lambda i: (i, 0)),
                   pl.BlockSpec((_R, _K), lambda i: (i, 0))],
        out_shape=[jax.ShapeDtypeStruct((_NTC, _K), f32),
                   jax.ShapeDtypeStruct((_NTC, _K), jnp.int32)],
        compiler_params=pltpu.CompilerParams(
            dimension_semantics=("arbitrary",)),
    )(sim2)

    mesh = plsc.VectorSubcoreMesh(core_axis_name="c", subcore_axis_name="s")
    sc_scores_flat, sc_idx_flat = pl.kernel(
        _sc_topk,
        out_type=[
            jax.ShapeDtypeStruct((_NSC * _K,), f32),
            jax.ShapeDtypeStruct((_NSC * _K,), jnp.int32),
        ],
        mesh=mesh,
        scratch_types=[
            pltpu.VMEM((_CH, _SEQ), f32),
            pltpu.VMEM((_NVEC + 32,), f32),
            pltpu.VMEM((_CH * _K + 16,), f32),
            pltpu.VMEM((_CH * _K + 16,), jnp.int32),
        ],
    )(sim2)

    scores = jnp.concatenate(
        [tc_scores, sc_scores_flat.reshape(_NSC, _K)], axis=0).reshape(B, S, _K)
    idx = jnp.concatenate(
        [tc_idx, sc_idx_flat.reshape(_NSC, _K)], axis=0).reshape(B, S, _K)
    minh = scores[..., :_K // 2]
    maxh = -scores[..., _K // 2:]
    return (emb, met, chrs, scores, idx, minh, maxh, ecc)


# speedup vs baseline: 2.2152x; 1.0287x over previous
import functools

import jax
import jax.numpy as jnp
from jax import lax
from jax.experimental import pallas as pl
from jax.experimental.pallas import tpu as pltpu
from jax.experimental.pallas import tpu_sc as plsc

_D_MODEL = 1024
_D_EMBED = 128
_N_CHR = 32
_ECC_BITS = 32
_K = 32
_SEQ = 2048
_R = 256
_NB = _SEQ // _R

_NC = 2
_NS = 16
_NW = _NC * _NS
_NROW = 4 * _SEQ
_NSC = 2816
_NTC = _NROW - _NSC
_RPW = _NSC // _NW
_CH = 8
_NCHUNK = _RPW // _CH
_NVEC = _SEQ // 16
_NEG = jnp.float32(-3e9)


def _tc_kernel(x_ref, we_ref, be_ref, wd_ref, wc_ref, bc_ref,
               wp_ref, bp_ref, w1_ref, b1_ref, w2_ref, b2_ref,
               emb_ref, met_ref, chr_ref, ecc_ref, sim_ref,
               embn_ref):
    p = pl.program_id(0)
    b = pl.program_id(1)
    s = pl.program_id(2)
    row0 = (b * _NB + s) * _R

    @pl.when(p == 0)
    def _phase0():
        x = x_ref[0]
        emb = jnp.dot(x, we_ref[...], preferred_element_type=jnp.float32) + be_ref[...]
        nrm = jnp.sqrt(jnp.sum(emb * emb, axis=1, keepdims=True)) + 1e-8
        embn_ref[pl.ds(row0, _R), :] = emb / nrm

    @pl.when(p == 1)
    def _phase1():
        x = x_ref[0]
        emb = jnp.dot(x, we_ref[...], preferred_element_type=jnp.float32) + be_ref[...]
        emb_ref[0] = emb
        met_ref[0] = jnp.dot(x.astype(jnp.bfloat16),
                             wd_ref[...].astype(jnp.bfloat16),
                             preferred_element_type=jnp.float32)
        chr_ref[0] = jnp.dot(x, wc_ref[...], preferred_element_type=jnp.float32) + bc_ref[...]
        pr = jnp.dot(x, wp_ref[...], preferred_element_type=jnp.float32) + bp_ref[...]
        h = jnp.tanh(jnp.dot(pr, w1_ref[...], preferred_element_type=jnp.float32) + b1_ref[...])
        ecc_ref[0] = jax.nn.sigmoid(
            jnp.dot(h, w2_ref[...], preferred_element_type=jnp.float32) + b2_ref[...])

        q = embn_ref[pl.ds(row0, _R), :]
        km = embn_ref[pl.ds(b * _SEQ, _SEQ), :]
        sim = jax.lax.dot_general(q, km, (((1,), (1,)), ((), ())),
                                  preferred_element_type=jnp.float32)
        rows = jax.lax.broadcasted_iota(jnp.int32, (_R, _SEQ), 0) + s * _R
        cols = jax.lax.broadcasted_iota(jnp.int32, (_R, _SEQ), 1)
        sim_ref[0] = jnp.where(rows == cols, jnp.float32(-1e9), sim)


def _rotate16(v, k):
    idx = lax.rem(lax.iota(jnp.int32, 16) + k, jnp.int32(16))
    dn = lax.GatherDimensionNumbers(offset_dims=(), collapsed_slice_dims=(0,),
                                    start_index_map=(0,))
    return lax.gather(v, idx.reshape(16, 1), dn, (1,),
                      mode=lax.GatherScatterMode.PROMISE_IN_BOUNDS)


def _splat_min(v):
    for k in (8, 4, 2, 1):
        v = jnp.minimum(v, _rotate16(v, k))
    return v


def _splat_max(v):
    for k in (8, 4, 2, 1):
        v = jnp.maximum(v, _rotate16(v, k))
    return v


def _sc_topk(sim_ref, outs_ref, outi_ref, chunk_ref, mbuf_ref,
             osc_ref, oix_ref):
    cid = lax.axis_index("c")
    sid = lax.axis_index("s")
    wid = sid * _NC + cid
    base = _NTC + wid * _RPW

    iota = lax.iota(jnp.int32, 16)
    iotaf = iota.astype(jnp.float32)
    big_f = jnp.full((16,), jnp.float32(1e9))
    negv = jnp.full((16,), _NEG)

    def chunk_body(c, _):
        pltpu.sync_copy(sim_ref.at[pl.ds(base + c * _CH, _CH)], chunk_ref)

        def row_body(rr, _):
            def p1(i, _):
                v = chunk_ref[rr, pl.ds(i * 16, 16)]
                mbuf_ref[pl.ds(i, 16)] = _splat_max(v)
                return 0
            lax.fori_loop(0, _NVEC, p1, 0)

            def extract(k, _):
                def fmax(j, acc):
                    return jnp.maximum(acc, mbuf_ref[pl.ds(j * 16, 16)])
                macc = lax.fori_loop(0, _NVEC // 16, fmax, negv)
                mkv = _splat_max(macc)

                def floc(j, vacc):
                    mv = mbuf_ref[pl.ds(j * 16, 16)]
                    loc = mv == mkv
                    cand = jnp.where(loc, iotaf + (j * 16).astype(jnp.float32),
                                     big_f)
                    return jnp.minimum(vacc, cand)
                vacc = lax.fori_loop(0, _NVEC // 16, floc, big_f)
                vecid = _splat_min(vacc)[0].astype(jnp.int32)

                w = chunk_ref[rr, pl.ds(vecid * 16, 16)]
                loc = w == mkv
                aminv = _splat_min(jnp.where(loc, iotaf, big_f)) \
                    + jnp.float32(16) * vecid.astype(jnp.float32)
                w2 = jnp.where(loc, negv, w)
                chunk_ref[rr, pl.ds(vecid * 16, 16)] = w2

                mwin = mbuf_ref[pl.ds(vecid, 16)]
                mbuf_ref[pl.ds(vecid, 16)] = jnp.where(
                    iota == 0, _splat_max(w2), mwin)

                opos = rr * _K + k
                osc_ref[pl.ds(opos, 16)] = mkv
                oix_ref[pl.ds(opos, 16)] = aminv.astype(jnp.int32)
                return 0
            lax.fori_loop(0, _K, extract, 0)
            return 0
        lax.fori_loop(0, _CH, row_body, 0)

        o0 = (wid * _RPW + c * _CH) * _K
        pltpu.sync_copy(osc_ref.at[pl.ds(0, _CH * _K)],
                        outs_ref.at[pl.ds(o0, _CH * _K)])
        pltpu.sync_copy(oix_ref.at[pl.ds(0, _CH * _K)],
                        outi_ref.at[pl.ds(o0, _CH * _K)])
        return 0
    lax.fori_loop(0, _NCHUNK, chunk_body, 0)


def _tc_topk(sim_ref, scores_ref, idx_ref):
    work = sim_ref[...]
    colsf = jax.lax.broadcasted_iota(jnp.int32, (_R, _SEQ), 1).astype(jnp.float32)
    s_list, i_list = [], []
    for _ in range(_K):
        m = jnp.max(work, axis=1, keepdims=True)
        loc = work == m
        amin = jnp.min(jnp.where(loc, colsf, jnp.float32(_SEQ)),
                       axis=1, keepdims=True)
        s_list.append(m)
        i_list.append(amin)
        work = jnp.where(loc, jnp.float32(-2e9), work)
    scores_ref[...] = jnp.concatenate(s_list, axis=1)
    idx_ref[...] = jnp.concatenate(i_list, axis=1).astype(jnp.int32)


def kernel(x, W_embed, b_embed, W_diag, W_chr, b_chr,
           W_ecc_proj, b_ecc_proj, W_e1, b_e1, W_e2, b_e2):
    B, S, D = x.shape
    f32 = jnp.float32

    be = b_embed.reshape(1, -1)
    bc = b_chr.reshape(1, -1)
    bp = b_ecc_proj.reshape(1, -1)
    b1 = b_e1.reshape(1, -1)
    b2 = b_e2.reshape(1, -1)

    full = lambda shp: pl.BlockSpec(shp, lambda p, b, s: (0,) * len(shp))
    blk = lambda w: pl.BlockSpec((1, _R, w), lambda p, b, s: (b, s, 0))
    emb, met, chrs, ecc, sim = pl.pallas_call(
        _tc_kernel,
        grid=(2, B, _NB),
        in_specs=[
            pl.BlockSpec((1, _R, D), lambda p, b, s: (b, s, 0)),
            full((D, _D_EMBED)), full((1, _D_EMBED)),
            full((D, D)),
            full((D, _N_CHR)), full((1, _N_CHR)),
            full((D, _ECC_BITS)), full((1, _ECC_BITS)),
            full((_ECC_BITS, 2 * _ECC_BITS)), full((1, 2 * _ECC_BITS)),
            full((2 * _ECC_BITS, _ECC_BITS)), full((1, _ECC_BITS)),
        ],
        out_specs=[
            blk(_D_EMBED), blk(D), blk(_N_CHR), blk(_ECC_BITS), blk(_SEQ),
        ],
        out_shape=[
            jax.ShapeDtypeStruct((B, S, _D_EMBED), f32),
            jax.ShapeDtypeStruct((B, S, D), f32),
            jax.ShapeDtypeStruct((B, S, _N_CHR), f32),
            jax.ShapeDtypeStruct((B, S, _ECC_BITS), f32),
            jax.ShapeDtypeStruct((B, S, _SEQ), f32),
        ],
        scratch_shapes=[pltpu.VMEM((B * S, _D_EMBED), f32)],
        compiler_params=pltpu.CompilerParams(
            dimension_semantics=("arbitrary", "arbitrary", "arbitrary")),
    )(x, W_embed, be, W_diag, W_chr, bc, W_ecc_proj, bp, W_e1, b1, W_e2, b2)

    sim2 = sim.reshape(B * S, S)

    tc_scores, tc_idx = pl.pallas_call(
        _tc_topk,
        grid=(_NTC // _R,),
        in_specs=[pl.BlockSpec((_R, _SEQ), lambda i: (i, 0))],
        out_specs=[pl.BlockSpec((_R, _K), lambda i: (i, 0)),
                   pl.BlockSpec((_R, _K), lambda i: (i, 0))],
        out_shape=[jax.ShapeDtypeStruct((_NTC, _K), f32),
                   jax.ShapeDtypeStruct((_NTC, _K), jnp.int32)],
        compiler_params=pltpu.CompilerParams(
            dimension_semantics=("arbitrary",)),
    )(sim2)

    mesh = plsc.VectorSubcoreMesh(core_axis_name="c", subcore_axis_name="s")
    sc_scores_flat, sc_idx_flat = pl.kernel(
        _sc_topk,
        out_type=[
            jax.ShapeDtypeStruct((_NSC * _K,), f32),
            jax.ShapeDtypeStruct((_NSC * _K,), jnp.int32),
        ],
        mesh=mesh,
        scratch_types=[
            pltpu.VMEM((_CH, _SEQ), f32),
            pltpu.VMEM((_NVEC + 32,), f32),
            pltpu.VMEM((_CH * _K + 16,), f32),
            pltpu.VMEM((_CH * _K + 16,), jnp.int32),
        ],
    )(sim2)

    scores = jnp.concatenate(
        [tc_scores, sc_scores_flat.reshape(_NSC, _K)], axis=0).reshape(B, S, _K)
    idx = jnp.concatenate(
        [tc_idx, sc_idx_flat.reshape(_NSC, _K)], axis=0).reshape(B, S, _K)
    minh = scores[..., :_K // 2]
    maxh = -scores[..., _K // 2:]
    return (emb, met, chrs, scores, idx, minh, maxh, ecc)
